# baseline (speedup 1.0000x reference)
import jax
import jax.numpy as jnp
from jax.experimental import pallas as pl


def kernel(density_grid, indices, densities):
    N = density_grid.shape[1]
    S = indices.shape[1]

    def per_cascade(grid_c, idx_c, d_c):
        i = jnp.arange(S, dtype=jnp.int32)
        winner = jax.ops.segment_max(i, idx_c, num_segments=N)
        has = winner >= 0
        dwin = d_c[jnp.clip(winner, 0, S - 1)]
        tmp = jnp.where(has, dwin, -1.0)
        return jnp.where((grid_c >= 0) & (tmp >= 0),
                         jnp.maximum(grid_c * 0.95, tmp), grid_c)

    updated = jax.vmap(per_cascade)(density_grid, indices, densities)
    return updated, updated.mean()


# speedup vs baseline: 42.0703x; 42.0703x over previous
import functools

import jax
import jax.numpy as jnp
from jax import lax
from jax.experimental import pallas as pl
from jax.experimental.pallas import tpu as pltpu
from jax.experimental.pallas import tpu_sc as plsc

NUM_CASCADES = 5
N_CELLS = 2097152
N_SAMPLES = 1048576
DECAY = 0.95
TOTAL = NUM_CASCADES * N_CELLS

NW = 32
CELLS_PER_W = N_CELLS // NW
CHUNK = 8192
VECS = CHUNK // 16
N_CHUNKS = N_SAMPLES // CHUNK

ROWS = 80
COLS = TOTAL // ROWS
BLK_R = 8


def _sc_scatter_body(idx_hbm, den_hbm, tmp_hbm, tmp_v, idx_v, den_v):
    cid = lax.axis_index("c")
    sid = lax.axis_index("s")
    wid = sid * 2 + cid
    base_cell = wid * CELLS_PER_W
    lane = lax.iota(jnp.int32, 16)
    nxt = jnp.minimum(lane + 1, 15)
    neg1 = jnp.full((16,), -1.0, jnp.float32)

    def cascade_body(c, carry):
        def init_body(j, carry):
            tmp_v[pl.ds(j * 16, 16)] = neg1
            return carry

        lax.fori_loop(0, CELLS_PER_W // 16, init_body, 0)

        def chunk_body(k, carry):
            off = c * N_SAMPLES + k * CHUNK
            pltpu.sync_copy(idx_hbm.at[pl.ds(off, CHUNK)], idx_v)
            pltpu.sync_copy(den_hbm.at[pl.ds(off, CHUNK)], den_v)

            def vec_body(v, carry):
                iv = idx_v[pl.ds(v * 16, 16)]
                dv = den_v[pl.ds(v * 16, 16)]
                belongs = (iv >> 16) == wid
                local = iv & 0xFFFF
                key = jnp.where(belongs, (local << 4) | lane, (1 << 20) + lane)
                sk, sd = plsc.sort_key_val(key, dv)
                so = sk >> 4
                so_next = so.at[nxt].get(mode="promise_in_bounds")
                mask = ((so != so_next) | (lane == 15)) & (so < CELLS_PER_W)
                so_safe = jnp.minimum(so, CELLS_PER_W - 1)
                plsc.store_scatter(tmp_v, [so_safe], sd, mask=mask)
                return carry

            lax.fori_loop(0, VECS, vec_body, 0)
            return carry

        lax.fori_loop(0, N_CHUNKS, chunk_body, 0)
        pltpu.sync_copy(
            tmp_v, tmp_hbm.at[pl.ds(c * N_CELLS + base_cell, CELLS_PER_W)])
        return carry

    lax.fori_loop(0, NUM_CASCADES, cascade_body, 0)


_sc_scatter = functools.partial(
    pl.kernel,
    out_type=jax.ShapeDtypeStruct((TOTAL,), jnp.float32),
    mesh=plsc.VectorSubcoreMesh(
        core_axis_name="c", subcore_axis_name="s",
        num_cores=2, num_subcores=16),
    compiler_params=pltpu.CompilerParams(needs_layout_passes=False),
    scratch_types=[
        pltpu.VMEM((CELLS_PER_W,), jnp.float32),
        pltpu.VMEM((CHUNK,), jnp.int32),
        pltpu.VMEM((CHUNK,), jnp.float32),
    ],
)(_sc_scatter_body)


def _merge_body(g_ref, t_ref, out_ref, mean_ref, acc_ref):
    i = pl.program_id(0)
    g = g_ref[...]
    t = t_ref[...]
    out = jnp.where((g >= 0.0) & (t >= 0.0), jnp.maximum(g * DECAY, t), g)
    out_ref[...] = out

    @pl.when(i == 0)
    def _():
        acc_ref[0, 0] = 0.0

    acc_ref[0, 0] += jnp.sum(out)

    @pl.when(i == pl.num_programs(0) - 1)
    def _():
        mean_ref[0, 0] = acc_ref[0, 0] / TOTAL


_merge = pl.pallas_call(
    _merge_body,
    grid=(ROWS // BLK_R,),
    in_specs=[
        pl.BlockSpec((BLK_R, COLS), lambda i: (i, 0)),
        pl.BlockSpec((BLK_R, COLS), lambda i: (i, 0)),
    ],
    out_specs=[
        pl.BlockSpec((BLK_R, COLS), lambda i: (i, 0)),
        pl.BlockSpec(memory_space=pltpu.SMEM),
    ],
    out_shape=[
        jax.ShapeDtypeStruct((ROWS, COLS), jnp.float32),
        jax.ShapeDtypeStruct((1, 1), jnp.float32),
    ],
    scratch_shapes=[pltpu.SMEM((1, 1), jnp.float32)],
)


def kernel(density_grid, indices, densities):
    idx_flat = indices.reshape(-1)
    den_flat = densities.reshape(-1)
    tmp = _sc_scatter(idx_flat, den_flat)
    upd, mean = _merge(density_grid.reshape(ROWS, COLS),
                       tmp.reshape(ROWS, COLS))
    return upd.reshape(NUM_CASCADES, N_CELLS), mean.reshape(())
